# baseline (speedup 1.0000x reference)
import jax
import jax.numpy as jnp
from jax.experimental import pallas as pl
from jax.experimental.pallas import tpu as pltpu

B, S, D = 4, 8192, 4096


def _body(x_hbm, o_ref, sem):
    copies = [
        pltpu.make_async_copy(x_hbm.at[b : b + 1, 0:1, :],
                              o_ref.at[b : b + 1], sem)
        for b in range(B)
    ]
    for c in copies:
        c.start()
    for c in copies:
        c.wait()


def kernel(hidden_state):
    return pl.pallas_call(
        _body,
        in_specs=[pl.BlockSpec(memory_space=pl.ANY)],
        out_shape=jax.ShapeDtypeStruct((B, 1, D), jnp.float32),
        scratch_shapes=[pltpu.SemaphoreType.DMA],
    )(hidden_state)
